# baseline (speedup 1.0000x reference)
import functools

import jax
import jax.numpy as jnp
from jax import lax
from jax.experimental import pallas as pl
from jax.experimental.pallas import tpu as pltpu
from jax.experimental.pallas import tpu_sc as plsc

N = 100000
N_IN = 16384
E = 3200000
E_FF = 1048576
D = 8
ALPHA = 0.9
RHO = 0.97
THR = 1.0
BETA = 1.8

NC = 2
NS = 16
NW = NC * NS

EPT_SMALL = 99968
CH = 16
CB = CH * 128
CH_LAST = 13
FPT = E_FF // NW

NPAD = 100352
SEG = NPAD // NS
SBN = D * N
SB_SEG = SBN // NS
SB_CH = 5000
NSB = SB_SEG // SB_CH

_mesh = plsc.VectorSubcoreMesh(
    core_axis_name="c", subcore_axis_name="s", num_cores=NC, num_subcores=NS)


@functools.partial(
    pl.kernel,
    out_type=jax.ShapeDtypeStruct((NC, NPAD), jnp.float32),
    mesh=_mesh,
    compiler_params=pltpu.CompilerParams(needs_layout_passes=False),
    scratch_types=[
        pltpu.VMEM_SHARED((SBN,), jnp.float32),
        pltpu.VMEM_SHARED((NPAD,), jnp.float32),
        pltpu.VMEM((N_IN,), jnp.float32),
        pltpu.VMEM((CB,), jnp.int32),
        pltpu.VMEM((CB,), jnp.int32),
        pltpu.VMEM((CB,), jnp.int32),
        pltpu.VMEM((CB,), jnp.int32),
        pltpu.VMEM((CB,), jnp.float32),
        pltpu.VMEM((CB,), jnp.float32),
        pltpu.VMEM((CB,), jnp.float32),
        pltpu.VMEM((CB,), jnp.float32),
        pltpu.VMEM((4, CH, 128), jnp.int32),
        pltpu.VMEM((2, CH, 128), jnp.int32),
        pltpu.VMEM((2, CH, 128), jnp.float32),
        pltpu.VMEM((2, CH, 128), jnp.float32),
        pltpu.VMEM((SEG,), jnp.float32),
        pltpu.VMEM((SB_CH,), jnp.float32),
        pltpu.VMEM((SB_CH,), jnp.float32),
        pltpu.SemaphoreType.DMA,
        pltpu.SemaphoreType.DMA,
        pltpu.SemaphoreType.DMA,
        pltpu.SemaphoreType.DMA,
        pltpu.SemaphoreType.DMA,
        pltpu.SemaphoreType.DMA,
        pltpu.SemaphoreType.DMA,
        pltpu.SemaphoreType.DMA,
        pltpu.SemaphoreType.DMA,
        pltpu.SemaphoreType.DMA,
        pltpu.SemaphoreType.DMA,
    ],
)
def _sc_currents(spike_h, pre_h, post_h, dly_h, ew_h, ff_h, fpre_h, fpost_h,
                 fw_h, out_h, spike_sh, acc_sh, ff_tab, pre0, pre1, dly0,
                 dly1, w0, w1, w2, w3, post_b, idx_b, xd_b, val_b, zbuf,
                 sbuf0, sbuf1, si0, si1, si2, si3, sem_g, ss0, ss1, sh0, sh1,
                 sem_v, sem_f):
    c = lax.axis_index("c")
    s = lax.axis_index("s")
    wid = s * NC + c
    pre_f = [pre0, pre1]
    dly_f = [dly0, dly1]
    w_f = [w0, w1, w2, w3]
    sem_in = [si0, si1, si2, si3]
    sem_s = [ss0, ss1]
    sem_h = [sh0, sh1]

    cp_ff = pltpu.async_copy(ff_h, ff_tab, sem_f)

    def _zero(i, carry):
        zbuf[pl.ds(i * 16, 16)] = jnp.zeros((16,), jnp.float32)
        return carry

    lax.fori_loop(0, SEG // 16, _zero, 0)
    pltpu.sync_copy(zbuf, acc_sh.at[pl.ds(s * SEG, SEG)])

    sbufs = [sbuf0, sbuf1]

    def _h2v(i):
        off = s * SB_SEG + i * SB_CH
        return pltpu.async_copy(spike_h.at[pl.ds(off, SB_CH)],
                                sbufs[i % 2], sem_h[i % 2])

    def _v2s(i):
        off = s * SB_SEG + i * SB_CH
        return pltpu.async_copy(sbufs[i % 2],
                                spike_sh.at[pl.ds(off, SB_CH)], sem_v)

    cps_h = {0: _h2v(0)}
    cps_v = {}
    for i in range(NSB):
        if i > 0:
            cps_v[i - 1].wait()
        if i + 1 < NSB:
            cps_h[i + 1] = _h2v(i + 1)
        cps_h[i].wait()
        cps_v[i] = _v2s(i)
    cps_v[NSB - 1].wait()
    cp_ff.wait()
    plsc.subcore_barrier()

    ebase = wid * EPT_SMALL + 128 * jnp.minimum(wid, 8)

    def _fire_staging(ch, k, r=CH):
        off = ebase + ch * CB
        fl = pl.ds(off, r * 128)
        pltpu.async_copy(pre_h.at[fl], pre_f[k % 2].at[pl.ds(0, r * 128)],
                         sem_in[k])
        pltpu.async_copy(dly_h.at[fl], dly_f[k % 2].at[pl.ds(0, r * 128)],
                         sem_in[k])
        pltpu.async_copy(ew_h.at[fl], w_f[k].at[pl.ds(0, r * 128)], sem_in[k])
        for j in range(r):
            pltpu.async_copy(post_h.at[pl.ds(off + j * 128, 128)],
                             post_b.at[k].at[j], sem_in[k])

    def _wait_staging(k, r=CH):
        fl = pl.ds(0, r * 128)
        pltpu.make_async_copy(pre_h.at[fl], pre_f[k % 2].at[fl],
                              sem_in[k]).wait()
        pltpu.make_async_copy(dly_h.at[fl], dly_f[k % 2].at[fl],
                              sem_in[k]).wait()
        pltpu.make_async_copy(ew_h.at[fl], w_f[k].at[fl], sem_in[k]).wait()
        for j in range(r):
            pltpu.make_async_copy(post_h.at[pl.ds(0, 128)],
                                  post_b.at[k].at[j], sem_in[k]).wait()

    def _fire_fstaging(ch, k):
        off = wid * FPT + ch * CB
        fl = pl.ds(off, CB)
        pltpu.async_copy(fpre_h.at[fl], pre_f[k % 2], sem_in[k])
        pltpu.async_copy(fw_h.at[fl], w_f[k], sem_in[k])
        for j in range(CH):
            pltpu.async_copy(fpost_h.at[pl.ds(off + j * 128, 128)],
                             post_b.at[k].at[j], sem_in[k])

    def _wait_fstaging(k):
        fl = pl.ds(0, CB)
        pltpu.make_async_copy(fpre_h.at[fl], pre_f[k % 2], sem_in[k]).wait()
        pltpu.make_async_copy(fw_h.at[fl], w_f[k], sem_in[k]).wait()
        for j in range(CH):
            pltpu.make_async_copy(fpost_h.at[pl.ds(0, 128)],
                                  post_b.at[k].at[j], sem_in[k]).wait()

    def _idx_row(k, j):
        for t in range(8):
            fl = pl.ds(j * 128 + t * 16, 16)
            dl = pl.ds(t * 16, 16)
            idx_b[k % 2, j, dl] = dly_f[k % 2][fl] * N + pre_f[k % 2][fl]

    def _idx_compute(k, r=CH):
        def body(j, carry):
            _idx_row(k, 2 * j)
            _idx_row(k, 2 * j + 1)
            return carry
        lax.fori_loop(0, r // 2, body, 0)
        if r % 2:
            _idx_row(k, r - 1)

    def _vals_row(k, j):
        for t in range(8):
            fl = pl.ds(j * 128 + t * 16, 16)
            dl = pl.ds(t * 16, 16)
            val_b[k % 2, j, dl] = w_f[k][fl] * xd_b[k % 2, j, dl]

    def _vals_compute(k, r=CH):
        def body(j, carry):
            _vals_row(k, 2 * j)
            _vals_row(k, 2 * j + 1)
            return carry
        lax.fori_loop(0, r // 2, body, 0)
        if r % 2:
            _vals_row(k, r - 1)

    def _fvals_compute(k):
        def body(j, carry):
            for jj in (2 * j, 2 * j + 1):
                for t in range(8):
                    fl = pl.ds(jj * 128 + t * 16, 16)
                    dl = pl.ds(t * 16, 16)
                    xv = plsc.load_gather(ff_tab, [pre_f[k % 2][fl]])
                    val_b[k % 2, jj, dl] = w_f[k][fl] * xv
            return carry
        lax.fori_loop(0, CH // 2, body, 0)

    def _fire_gathers(k, r=CH):
        for j in range(r):
            pltpu.async_copy(spike_sh.at[idx_b.at[k % 2].at[j]],
                             xd_b.at[k % 2].at[j], sem_g)

    def _wait_gathers(k, r=CH):
        for j in range(r):
            pltpu.make_async_copy(spike_h.at[pl.ds(0, 128)],
                                  xd_b.at[k % 2].at[j], sem_g).wait()

    def _fire_scatters(k, p, r=CH):
        for j in range(r):
            pltpu.async_copy(val_b.at[k % 2].at[j],
                             acc_sh.at[post_b.at[k].at[j]], sem_s[p],
                             add=True)

    def _wait_scatters(k, p, r=CH):
        for j in range(r):
            pltpu.make_async_copy(val_b.at[k % 2].at[j],
                                  acc_sh.at[post_b.at[k].at[j]],
                                  sem_s[p]).wait()

    _fire_staging(0, 0)
    _fire_staging(1, 1)

    def _main_body(i, carry):
        for k in range(4):
            ch = 4 * i + k
            _wait_staging(k)
            _idx_compute(k)
            pk, pp = (k - 1) % 4, (k - 1) % 2
            if k > 0:
                _wait_gathers(pk)
                _vals_compute(pk)
                _fire_scatters(pk, pp)
            else:
                @pl.when(i > 0)
                def _():
                    _wait_gathers(3)
                    _vals_compute(3)
                    _fire_scatters(3, 1)
            _fire_gathers(k)
            if k >= 2:
                _wait_scatters(k - 2, k % 2)
            else:
                @pl.when(i > 0)
                def _():
                    _wait_scatters(k + 2, k % 2)
            if k == 2:
                @pl.when(i < 11)
                def _():
                    _fire_staging(ch + 2, 0)

                @pl.when(i == 11)
                def _():
                    _fire_staging(ch + 2, 0, CH_LAST)
            elif k == 3:
                @pl.when(i < 11)
                def _():
                    _fire_staging(ch + 2, 1)
            else:
                _fire_staging(ch + 2, k + 2)
        return carry

    lax.fori_loop(0, 12, _main_body, 0)

    _wait_staging(0, CH_LAST)
    _idx_compute(0, CH_LAST)
    _wait_gathers(3)
    _vals_compute(3)
    _fire_scatters(3, 1)
    _fire_gathers(0, CH_LAST)
    _wait_scatters(2, 0)
    _wait_gathers(0, CH_LAST)
    _vals_compute(0, CH_LAST)
    _fire_scatters(0, 0, CH_LAST)
    _wait_scatters(3, 1)
    _wait_scatters(0, 0, CH_LAST)

    @pl.when(wid < 8)
    def _():
        sl = pl.ds(ebase + 48 * CB + CH_LAST * 128, 128)
        d128 = pl.ds(0, 128)
        pltpu.sync_copy(pre_h.at[sl], pre_f[0].at[d128])
        pltpu.sync_copy(dly_h.at[sl], dly_f[0].at[d128])
        pltpu.sync_copy(ew_h.at[sl], w_f[0].at[d128])
        pltpu.sync_copy(post_h.at[sl], post_b.at[0].at[0])
        for t in range(8):
            dl = pl.ds(t * 16, 16)
            idx_b[0, 0, dl] = dly_f[0][dl] * N + pre_f[0][dl]
        pltpu.sync_copy(spike_sh.at[idx_b.at[0].at[0]], xd_b.at[0].at[0])
        for t in range(8):
            dl = pl.ds(t * 16, 16)
            val_b[0, 0, dl] = w_f[0][dl] * xd_b[0, 0, dl]
        pltpu.sync_copy(val_b.at[0].at[0], acc_sh.at[post_b.at[0].at[0]],
                        add=True)

    _fire_fstaging(0, 0)
    _fire_fstaging(1, 1)

    def _ff_body(i, carry):
        for k in range(4):
            ch = 4 * i + k
            _wait_fstaging(k)
            if k >= 2:
                _wait_scatters(k - 2, k % 2)
            else:
                @pl.when(i > 0)
                def _():
                    _wait_scatters(k + 2, k % 2)
            _fvals_compute(k)
            _fire_scatters(k, k % 2)
            if k >= 2:
                @pl.when(i < 3)
                def _():
                    _fire_fstaging(ch + 2, (k + 2) % 4)
            else:
                _fire_fstaging(ch + 2, (k + 2) % 4)
        return carry

    lax.fori_loop(0, 4, _ff_body, 0)
    _wait_scatters(2, 0)
    _wait_scatters(3, 1)

    plsc.subcore_barrier()
    pltpu.sync_copy(acc_sh.at[pl.ds(s * SEG, SEG)], zbuf)
    pltpu.sync_copy(zbuf, out_h.at[c, pl.ds(s * SEG, SEG)])


def _alif_update(i_ref, v_ref, a_ref, x_out, v_out, a_out):
    cur = i_ref[0, pl.ds(0, N)] + i_ref[1, pl.ds(0, N)]
    V = v_ref[...]
    A = a_ref[...]
    X = (V - (THR + BETA * A) >= 0.0).astype(jnp.float32)
    x_out[...] = X
    v_out[...] = ALPHA * V * (1.0 - X) + cur
    a_out[...] = RHO * A + X


def kernel(FF, V, a, spike_buffer, edge_w, ff_w, edge_pre, edge_post,
           edge_delay, ff_pre, ff_post):
    sb_flat = spike_buffer.reshape(-1)

    acc = _sc_currents(sb_flat, edge_pre, edge_post, edge_delay, edge_w, FF,
                       ff_pre, ff_post, ff_w)

    shp = jax.ShapeDtypeStruct((N,), jnp.float32)
    X, Vn, an = pl.pallas_call(
        _alif_update,
        out_shape=[shp, shp, shp],
    )(acc, V, a)
    return (X, Vn, an)


# speedup vs baseline: 1.0249x; 1.0249x over previous
import functools

import jax
import jax.numpy as jnp
from jax import lax
from jax.experimental import pallas as pl
from jax.experimental.pallas import tpu as pltpu
from jax.experimental.pallas import tpu_sc as plsc

N = 100000
N_IN = 16384
E = 3200000
E_FF = 1048576
D = 8
ALPHA = 0.9
RHO = 0.97
THR = 1.0
BETA = 1.8

NC = 2
NS = 16
NW = NC * NS

EPT_SMALL = 99968
CH = 16
CB = CH * 128
CH_LAST = 13
FPT = E_FF // NW

NPAD = 100352
SEG = NPAD // NS
SBN = D * N
SB_SEG = SBN // NS
SB_CH = 5000
NSB = SB_SEG // SB_CH

_mesh = plsc.VectorSubcoreMesh(
    core_axis_name="c", subcore_axis_name="s", num_cores=NC, num_subcores=NS)


@functools.partial(
    pl.kernel,
    out_type=jax.ShapeDtypeStruct((NC, NPAD), jnp.float32),
    mesh=_mesh,
    compiler_params=pltpu.CompilerParams(needs_layout_passes=False),
    scratch_types=[
        pltpu.VMEM_SHARED((SBN,), jnp.float32),
        pltpu.VMEM_SHARED((NPAD,), jnp.float32),
        pltpu.VMEM((N_IN,), jnp.float32),
        pltpu.VMEM((CB,), jnp.int32),
        pltpu.VMEM((CB,), jnp.int32),
        pltpu.VMEM((CB,), jnp.int32),
        pltpu.VMEM((CB,), jnp.int32),
        pltpu.VMEM((CB,), jnp.float32),
        pltpu.VMEM((CB,), jnp.float32),
        pltpu.VMEM((CB,), jnp.float32),
        pltpu.VMEM((CB,), jnp.float32),
        pltpu.VMEM((4, CH, 128), jnp.int32),
        pltpu.VMEM((2, CH, 128), jnp.int32),
        pltpu.VMEM((2, CH, 128), jnp.float32),
        pltpu.VMEM((2, CH, 128), jnp.float32),
        pltpu.VMEM((SEG,), jnp.float32),
        pltpu.VMEM((SB_CH,), jnp.float32),
        pltpu.VMEM((SB_CH,), jnp.float32),
        pltpu.SemaphoreType.DMA,
        pltpu.SemaphoreType.DMA,
        pltpu.SemaphoreType.DMA,
        pltpu.SemaphoreType.DMA,
        pltpu.SemaphoreType.DMA,
        pltpu.SemaphoreType.DMA,
        pltpu.SemaphoreType.DMA,
        pltpu.SemaphoreType.DMA,
        pltpu.SemaphoreType.DMA,
        pltpu.SemaphoreType.DMA,
        pltpu.SemaphoreType.DMA,
    ],
)
def _sc_currents(spike_h, pre_h, post_h, dly_h, ew_h, ff_h, fpre_h, fpost_h,
                 fw_h, out_h, spike_sh, acc_sh, ff_tab, pre0, pre1, dly0,
                 dly1, w0, w1, w2, w3, post_b, idx_b, xd_b, val_b, zbuf,
                 sbuf0, sbuf1, si0, si1, si2, si3, sem_g, ss0, ss1, sh0, sh1,
                 sem_v, sem_f):
    c = lax.axis_index("c")
    s = lax.axis_index("s")
    wid = s * NC + c
    pre_f = [pre0, pre1]
    dly_f = [dly0, dly1]
    w_f = [w0, w1, w2, w3]
    sem_in = [si0, si1, si2, si3]
    sem_s = [ss0, ss1]
    sem_h = [sh0, sh1]

    cp_ff = pltpu.async_copy(ff_h, ff_tab, sem_f)

    def _zero(i, carry):
        zbuf[pl.ds(i * 16, 16)] = jnp.zeros((16,), jnp.float32)
        return carry

    lax.fori_loop(0, SEG // 16, _zero, 0)
    pltpu.sync_copy(zbuf, acc_sh.at[pl.ds(s * SEG, SEG)])

    sbufs = [sbuf0, sbuf1]

    def _h2v(i):
        off = s * SB_SEG + i * SB_CH
        return pltpu.async_copy(spike_h.at[pl.ds(off, SB_CH)],
                                sbufs[i % 2], sem_h[i % 2])

    def _v2s(i):
        off = s * SB_SEG + i * SB_CH
        return pltpu.async_copy(sbufs[i % 2],
                                spike_sh.at[pl.ds(off, SB_CH)], sem_v)

    cps_h = {0: _h2v(0)}
    cps_v = {}
    for i in range(NSB):
        if i > 0:
            cps_v[i - 1].wait()
        if i + 1 < NSB:
            cps_h[i + 1] = _h2v(i + 1)
        cps_h[i].wait()
        cps_v[i] = _v2s(i)
    cps_v[NSB - 1].wait()
    cp_ff.wait()
    plsc.subcore_barrier()

    ebase = wid * EPT_SMALL + 128 * jnp.minimum(wid, 8)

    def _fire_staging(ch, k, r=CH):
        off = ebase + ch * CB
        fl = pl.ds(off, r * 128)
        pltpu.async_copy(pre_h.at[fl], pre_f[k % 2].at[pl.ds(0, r * 128)],
                         sem_in[k])
        pltpu.async_copy(dly_h.at[fl], dly_f[k % 2].at[pl.ds(0, r * 128)],
                         sem_in[k])
        pltpu.async_copy(ew_h.at[fl], w_f[k].at[pl.ds(0, r * 128)], sem_in[k])
        for j in range(r):
            pltpu.async_copy(post_h.at[pl.ds(off + j * 128, 128)],
                             post_b.at[k].at[j], sem_in[k])

    def _wait_staging(k, r=CH):
        fl = pl.ds(0, r * 128)
        pltpu.make_async_copy(pre_h.at[fl], pre_f[k % 2].at[fl],
                              sem_in[k]).wait()
        pltpu.make_async_copy(dly_h.at[fl], dly_f[k % 2].at[fl],
                              sem_in[k]).wait()
        pltpu.make_async_copy(ew_h.at[fl], w_f[k].at[fl], sem_in[k]).wait()
        for j in range(r):
            pltpu.make_async_copy(post_h.at[pl.ds(0, 128)],
                                  post_b.at[k].at[j], sem_in[k]).wait()

    def _fire_fstaging(ch, k):
        off = wid * FPT + ch * CB
        fl = pl.ds(off, CB)
        pltpu.async_copy(fpre_h.at[fl], pre_f[k % 2], sem_in[k])
        pltpu.async_copy(fw_h.at[fl], w_f[k], sem_in[k])
        for j in range(CH):
            pltpu.async_copy(fpost_h.at[pl.ds(off + j * 128, 128)],
                             post_b.at[k].at[j], sem_in[k])

    def _wait_fstaging(k):
        fl = pl.ds(0, CB)
        pltpu.make_async_copy(fpre_h.at[fl], pre_f[k % 2], sem_in[k]).wait()
        pltpu.make_async_copy(fw_h.at[fl], w_f[k], sem_in[k]).wait()
        for j in range(CH):
            pltpu.make_async_copy(fpost_h.at[pl.ds(0, 128)],
                                  post_b.at[k].at[j], sem_in[k]).wait()

    def _idx_compute(k, r=CH):
        def body(j, carry):
            for t in range(8):
                fl = pl.ds(j * 128 + t * 16, 16)
                dl = pl.ds(t * 16, 16)
                idx_b[k % 2, j, dl] = dly_f[k % 2][fl] * N + pre_f[k % 2][fl]
            return carry
        lax.fori_loop(0, r, body, 0)

    def _vals_compute(k, r=CH):
        def body(j, carry):
            for t in range(8):
                fl = pl.ds(j * 128 + t * 16, 16)
                dl = pl.ds(t * 16, 16)
                val_b[k % 2, j, dl] = w_f[k][fl] * xd_b[k % 2, j, dl]
            return carry
        lax.fori_loop(0, r, body, 0)

    def _fvals_compute(k):
        def body(j, carry):
            for t in range(8):
                fl = pl.ds(j * 128 + t * 16, 16)
                dl = pl.ds(t * 16, 16)
                xv = plsc.load_gather(ff_tab, [pre_f[k % 2][fl]])
                val_b[k % 2, j, dl] = w_f[k][fl] * xv
            return carry
        lax.fori_loop(0, CH, body, 0)

    def _fire_gathers(k, r=CH):
        for j in range(r):
            pltpu.async_copy(spike_sh.at[idx_b.at[k % 2].at[j]],
                             xd_b.at[k % 2].at[j], sem_g)

    def _wait_gathers(k, r=CH):
        for j in range(r):
            pltpu.make_async_copy(spike_h.at[pl.ds(0, 128)],
                                  xd_b.at[k % 2].at[j], sem_g).wait()

    def _fire_scatters(k, p, r=CH):
        for j in range(r):
            pltpu.async_copy(val_b.at[k % 2].at[j],
                             acc_sh.at[post_b.at[k].at[j]], sem_s[p],
                             add=True)

    def _wait_scatters(k, p, r=CH):
        for j in range(r):
            pltpu.make_async_copy(val_b.at[k % 2].at[j],
                                  acc_sh.at[post_b.at[k].at[j]],
                                  sem_s[p]).wait()

    _fire_staging(0, 0)
    _fire_staging(1, 1)

    def _main_body(i, carry):
        for k in range(4):
            ch = 4 * i + k
            _wait_staging(k)
            _idx_compute(k)
            pk, pp = (k - 1) % 4, (k - 1) % 2
            if k > 0:
                _wait_gathers(pk)
                _vals_compute(pk)
                _fire_scatters(pk, pp)
            else:
                @pl.when(i > 0)
                def _():
                    _wait_gathers(3)
                    _vals_compute(3)
                    _fire_scatters(3, 1)
            _fire_gathers(k)
            if k >= 2:
                _wait_scatters(k - 2, k % 2)
            else:
                @pl.when(i > 0)
                def _():
                    _wait_scatters(k + 2, k % 2)
            if k == 2:
                @pl.when(i < 11)
                def _():
                    _fire_staging(ch + 2, 0)

                @pl.when(i == 11)
                def _():
                    _fire_staging(ch + 2, 0, CH_LAST)
            elif k == 3:
                @pl.when(i < 11)
                def _():
                    _fire_staging(ch + 2, 1)
            else:
                _fire_staging(ch + 2, k + 2)
        return carry

    lax.fori_loop(0, 12, _main_body, 0)

    _wait_staging(0, CH_LAST)
    _idx_compute(0, CH_LAST)
    _wait_gathers(3)
    _vals_compute(3)
    _fire_scatters(3, 1)
    _fire_gathers(0, CH_LAST)
    _wait_scatters(2, 0)
    _wait_gathers(0, CH_LAST)
    _vals_compute(0, CH_LAST)
    _fire_scatters(0, 0, CH_LAST)
    _wait_scatters(3, 1)
    _wait_scatters(0, 0, CH_LAST)

    @pl.when(wid < 8)
    def _():
        sl = pl.ds(ebase + 48 * CB + CH_LAST * 128, 128)
        d128 = pl.ds(0, 128)
        pltpu.sync_copy(pre_h.at[sl], pre_f[0].at[d128])
        pltpu.sync_copy(dly_h.at[sl], dly_f[0].at[d128])
        pltpu.sync_copy(ew_h.at[sl], w_f[0].at[d128])
        pltpu.sync_copy(post_h.at[sl], post_b.at[0].at[0])
        for t in range(8):
            dl = pl.ds(t * 16, 16)
            idx_b[0, 0, dl] = dly_f[0][dl] * N + pre_f[0][dl]
        pltpu.sync_copy(spike_sh.at[idx_b.at[0].at[0]], xd_b.at[0].at[0])
        for t in range(8):
            dl = pl.ds(t * 16, 16)
            val_b[0, 0, dl] = w_f[0][dl] * xd_b[0, 0, dl]
        pltpu.sync_copy(val_b.at[0].at[0], acc_sh.at[post_b.at[0].at[0]],
                        add=True)

    _fire_fstaging(0, 0)
    _fire_fstaging(1, 1)

    def _ff_body(i, carry):
        for k in range(4):
            ch = 4 * i + k
            _wait_fstaging(k)
            if k >= 2:
                _wait_scatters(k - 2, k % 2)
            else:
                @pl.when(i > 0)
                def _():
                    _wait_scatters(k + 2, k % 2)
            _fvals_compute(k)
            _fire_scatters(k, k % 2)
            if k >= 2:
                @pl.when(i < 3)
                def _():
                    _fire_fstaging(ch + 2, (k + 2) % 4)
            else:
                _fire_fstaging(ch + 2, (k + 2) % 4)
        return carry

    lax.fori_loop(0, 4, _ff_body, 0)
    _wait_scatters(2, 0)
    _wait_scatters(3, 1)

    plsc.subcore_barrier()
    pltpu.sync_copy(acc_sh.at[pl.ds(s * SEG, SEG)], zbuf)
    pltpu.sync_copy(zbuf, out_h.at[c, pl.ds(s * SEG, SEG)])


def _alif_update(i_ref, v_ref, a_ref, x_out, v_out, a_out):
    cur = i_ref[0, pl.ds(0, N)] + i_ref[1, pl.ds(0, N)]
    V = v_ref[...]
    A = a_ref[...]
    X = (V - (THR + BETA * A) >= 0.0).astype(jnp.float32)
    x_out[...] = X
    v_out[...] = ALPHA * V * (1.0 - X) + cur
    a_out[...] = RHO * A + X


def kernel(FF, V, a, spike_buffer, edge_w, ff_w, edge_pre, edge_post,
           edge_delay, ff_pre, ff_post):
    sb_flat = spike_buffer.reshape(-1)

    acc = _sc_currents(sb_flat, edge_pre, edge_post, edge_delay, edge_w, FF,
                       ff_pre, ff_post, ff_w)

    shp = jax.ShapeDtypeStruct((N,), jnp.float32)
    X, Vn, an = pl.pallas_call(
        _alif_update,
        out_shape=[shp, shp, shp],
    )(acc, V, a)
    return (X, Vn, an)


# speedup vs baseline: 1.0270x; 1.0021x over previous
import functools

import jax
import jax.numpy as jnp
from jax import lax
from jax.experimental import pallas as pl
from jax.experimental.pallas import tpu as pltpu
from jax.experimental.pallas import tpu_sc as plsc

N = 100000
N_IN = 16384
E = 3200000
E_FF = 1048576
D = 8
ALPHA = 0.9
RHO = 0.97
THR = 1.0
BETA = 1.8

NC = 2
NS = 16
NW = NC * NS

EPT_SMALL = 99968
CH = 16
CB = CH * 128
CH_LAST = 13
FPT = E_FF // NW

NPAD = 100352
SEG = NPAD // NS
SBN = D * N
SB_SEG = SBN // NS
SB_CH = 5000
NSB = SB_SEG // SB_CH

_mesh = plsc.VectorSubcoreMesh(
    core_axis_name="c", subcore_axis_name="s", num_cores=NC, num_subcores=NS)


@functools.partial(
    pl.kernel,
    out_type=jax.ShapeDtypeStruct((NC, NPAD), jnp.float32),
    mesh=_mesh,
    compiler_params=pltpu.CompilerParams(needs_layout_passes=False),
    scratch_types=[
        pltpu.VMEM_SHARED((SBN,), jnp.float32),
        pltpu.VMEM_SHARED((NPAD,), jnp.float32),
        pltpu.VMEM((N_IN,), jnp.float32),
        pltpu.VMEM((CB,), jnp.int32),
        pltpu.VMEM((CB,), jnp.int32),
        pltpu.VMEM((CB,), jnp.int32),
        pltpu.VMEM((CB,), jnp.int32),
        pltpu.VMEM((CB,), jnp.float32),
        pltpu.VMEM((CB,), jnp.float32),
        pltpu.VMEM((CB,), jnp.float32),
        pltpu.VMEM((CB,), jnp.float32),
        pltpu.VMEM((4, CH, 128), jnp.int32),
        pltpu.VMEM((2, CH, 128), jnp.int32),
        pltpu.VMEM((2, CH, 128), jnp.float32),
        pltpu.VMEM((2, CH, 128), jnp.float32),
        pltpu.VMEM((SEG,), jnp.float32),
        pltpu.VMEM((SB_CH,), jnp.float32),
        pltpu.VMEM((SB_CH,), jnp.float32),
        pltpu.SemaphoreType.DMA,
        pltpu.SemaphoreType.DMA,
        pltpu.SemaphoreType.DMA,
        pltpu.SemaphoreType.DMA,
        pltpu.SemaphoreType.DMA,
        pltpu.SemaphoreType.DMA,
        pltpu.SemaphoreType.DMA,
        pltpu.SemaphoreType.DMA,
        pltpu.SemaphoreType.DMA,
        pltpu.SemaphoreType.DMA,
        pltpu.SemaphoreType.DMA,
    ],
)
def _sc_currents(spike_h, pre_h, post_h, dly_h, ew_h, ff_h, fpre_h, fpost_h,
                 fw_h, out_h, spike_sh, acc_sh, ff_tab, pre0, pre1, dly0,
                 dly1, w0, w1, w2, w3, post_b, idx_b, xd_b, val_b, zbuf,
                 sbuf0, sbuf1, si0, si1, si2, si3, sem_g, ss0, ss1, sh0, sh1,
                 sem_v, sem_f):
    c = lax.axis_index("c")
    s = lax.axis_index("s")
    wid = s * NC + c
    pre_f = [pre0, pre1]
    dly_f = [dly0, dly1]
    w_f = [w0, w1, w2, w3]
    sem_in = [si0, si1, si2, si3]
    sem_s = [ss0, ss1]
    sem_h = [sh0, sh1]

    ebase = wid * EPT_SMALL + 128 * jnp.minimum(wid, 8)

    def _fire_staging(ch, k, r=CH):
        off = ebase + ch * CB
        fl = pl.ds(off, r * 128)
        pltpu.async_copy(pre_h.at[fl], pre_f[k % 2].at[pl.ds(0, r * 128)],
                         sem_in[k])
        pltpu.async_copy(dly_h.at[fl], dly_f[k % 2].at[pl.ds(0, r * 128)],
                         sem_in[k])
        pltpu.async_copy(ew_h.at[fl], w_f[k].at[pl.ds(0, r * 128)], sem_in[k])
        for j in range(r):
            pltpu.async_copy(post_h.at[pl.ds(off + j * 128, 128)],
                             post_b.at[k].at[j], sem_in[k])

    def _wait_staging(k, r=CH):
        fl = pl.ds(0, r * 128)
        pltpu.make_async_copy(pre_h.at[fl], pre_f[k % 2].at[fl],
                              sem_in[k]).wait()
        pltpu.make_async_copy(dly_h.at[fl], dly_f[k % 2].at[fl],
                              sem_in[k]).wait()
        pltpu.make_async_copy(ew_h.at[fl], w_f[k].at[fl], sem_in[k]).wait()
        for j in range(r):
            pltpu.make_async_copy(post_h.at[pl.ds(0, 128)],
                                  post_b.at[k].at[j], sem_in[k]).wait()

    def _fire_fstaging(ch, k):
        off = wid * FPT + ch * CB
        fl = pl.ds(off, CB)
        pltpu.async_copy(fpre_h.at[fl], pre_f[k % 2], sem_in[k])
        pltpu.async_copy(fw_h.at[fl], w_f[k], sem_in[k])
        for j in range(CH):
            pltpu.async_copy(fpost_h.at[pl.ds(off + j * 128, 128)],
                             post_b.at[k].at[j], sem_in[k])

    def _wait_fstaging(k):
        fl = pl.ds(0, CB)
        pltpu.make_async_copy(fpre_h.at[fl], pre_f[k % 2], sem_in[k]).wait()
        pltpu.make_async_copy(fw_h.at[fl], w_f[k], sem_in[k]).wait()
        for j in range(CH):
            pltpu.make_async_copy(fpost_h.at[pl.ds(0, 128)],
                                  post_b.at[k].at[j], sem_in[k]).wait()

    def _idx_compute(k, r=CH):
        def body(j, carry):
            for t in range(8):
                fl = pl.ds(j * 128 + t * 16, 16)
                dl = pl.ds(t * 16, 16)
                idx_b[k % 2, j, dl] = dly_f[k % 2][fl] * N + pre_f[k % 2][fl]
            return carry
        lax.fori_loop(0, r, body, 0)

    def _vals_compute(k, r=CH):
        def body(j, carry):
            for t in range(8):
                fl = pl.ds(j * 128 + t * 16, 16)
                dl = pl.ds(t * 16, 16)
                val_b[k % 2, j, dl] = w_f[k][fl] * xd_b[k % 2, j, dl]
            return carry
        lax.fori_loop(0, r, body, 0)

    def _fvals_compute(k):
        def body(j, carry):
            for t in range(8):
                fl = pl.ds(j * 128 + t * 16, 16)
                dl = pl.ds(t * 16, 16)
                xv = plsc.load_gather(ff_tab, [pre_f[k % 2][fl]])
                val_b[k % 2, j, dl] = w_f[k][fl] * xv
            return carry
        lax.fori_loop(0, CH, body, 0)

    def _fire_gathers(k, r=CH):
        for j in range(r):
            pltpu.async_copy(spike_sh.at[idx_b.at[k % 2].at[j]],
                             xd_b.at[k % 2].at[j], sem_g)

    def _wait_gathers(k, r=CH):
        for j in range(r):
            pltpu.make_async_copy(spike_h.at[pl.ds(0, 128)],
                                  xd_b.at[k % 2].at[j], sem_g).wait()

    def _fire_scatters(k, p, r=CH):
        for j in range(r):
            pltpu.async_copy(val_b.at[k % 2].at[j],
                             acc_sh.at[post_b.at[k].at[j]], sem_s[p],
                             add=True)

    def _wait_scatters(k, p, r=CH):
        for j in range(r):
            pltpu.make_async_copy(val_b.at[k % 2].at[j],
                                  acc_sh.at[post_b.at[k].at[j]],
                                  sem_s[p]).wait()

    _fire_staging(0, 0)
    _fire_staging(1, 1)
    cp_ff = pltpu.async_copy(ff_h, ff_tab, sem_f)

    def _zero(i, carry):
        zbuf[pl.ds(i * 16, 16)] = jnp.zeros((16,), jnp.float32)
        return carry

    lax.fori_loop(0, SEG // 16, _zero, 0)
    pltpu.sync_copy(zbuf, acc_sh.at[pl.ds(s * SEG, SEG)])

    sbufs = [sbuf0, sbuf1]

    def _h2v(i):
        off = s * SB_SEG + i * SB_CH
        return pltpu.async_copy(spike_h.at[pl.ds(off, SB_CH)],
                                sbufs[i % 2], sem_h[i % 2])

    def _v2s(i):
        off = s * SB_SEG + i * SB_CH
        return pltpu.async_copy(sbufs[i % 2],
                                spike_sh.at[pl.ds(off, SB_CH)], sem_v)

    cps_h = {0: _h2v(0)}
    cps_v = {}
    for i in range(NSB):
        if i > 0:
            cps_v[i - 1].wait()
        if i + 1 < NSB:
            cps_h[i + 1] = _h2v(i + 1)
        cps_h[i].wait()
        cps_v[i] = _v2s(i)
    cps_v[NSB - 1].wait()
    cp_ff.wait()
    plsc.subcore_barrier()

    def _main_body(i, carry):
        for k in range(4):
            ch = 4 * i + k
            _wait_staging(k)
            _idx_compute(k)
            pk, pp = (k - 1) % 4, (k - 1) % 2
            if k > 0:
                _wait_gathers(pk)
                _vals_compute(pk)
                _fire_scatters(pk, pp)
            else:
                @pl.when(i > 0)
                def _():
                    _wait_gathers(3)
                    _vals_compute(3)
                    _fire_scatters(3, 1)
            _fire_gathers(k)
            if k >= 2:
                _wait_scatters(k - 2, k % 2)
            else:
                @pl.when(i > 0)
                def _():
                    _wait_scatters(k + 2, k % 2)
            if k == 2:
                @pl.when(i < 11)
                def _():
                    _fire_staging(ch + 2, 0)

                @pl.when(i == 11)
                def _():
                    _fire_staging(ch + 2, 0, CH_LAST)
            elif k == 3:
                @pl.when(i < 11)
                def _():
                    _fire_staging(ch + 2, 1)
            else:
                _fire_staging(ch + 2, k + 2)
        return carry

    lax.fori_loop(0, 12, _main_body, 0)

    _wait_staging(0, CH_LAST)
    _idx_compute(0, CH_LAST)
    _wait_gathers(3)
    _vals_compute(3)
    _fire_scatters(3, 1)
    _fire_gathers(0, CH_LAST)
    _wait_scatters(2, 0)
    _wait_gathers(0, CH_LAST)
    _vals_compute(0, CH_LAST)
    _fire_scatters(0, 0, CH_LAST)
    _wait_scatters(3, 1)
    _wait_scatters(0, 0, CH_LAST)

    @pl.when(wid < 8)
    def _():
        sl = pl.ds(ebase + 48 * CB + CH_LAST * 128, 128)
        d128 = pl.ds(0, 128)
        pltpu.sync_copy(pre_h.at[sl], pre_f[0].at[d128])
        pltpu.sync_copy(dly_h.at[sl], dly_f[0].at[d128])
        pltpu.sync_copy(ew_h.at[sl], w_f[0].at[d128])
        pltpu.sync_copy(post_h.at[sl], post_b.at[0].at[0])
        for t in range(8):
            dl = pl.ds(t * 16, 16)
            idx_b[0, 0, dl] = dly_f[0][dl] * N + pre_f[0][dl]
        pltpu.sync_copy(spike_sh.at[idx_b.at[0].at[0]], xd_b.at[0].at[0])
        for t in range(8):
            dl = pl.ds(t * 16, 16)
            val_b[0, 0, dl] = w_f[0][dl] * xd_b[0, 0, dl]
        pltpu.sync_copy(val_b.at[0].at[0], acc_sh.at[post_b.at[0].at[0]],
                        add=True)

    _fire_fstaging(0, 0)
    _fire_fstaging(1, 1)

    def _ff_body(i, carry):
        for k in range(4):
            ch = 4 * i + k
            _wait_fstaging(k)
            if k >= 2:
                _wait_scatters(k - 2, k % 2)
            else:
                @pl.when(i > 0)
                def _():
                    _wait_scatters(k + 2, k % 2)
            _fvals_compute(k)
            _fire_scatters(k, k % 2)
            if k >= 2:
                @pl.when(i < 3)
                def _():
                    _fire_fstaging(ch + 2, (k + 2) % 4)
            else:
                _fire_fstaging(ch + 2, (k + 2) % 4)
        return carry

    lax.fori_loop(0, 4, _ff_body, 0)
    _wait_scatters(2, 0)
    _wait_scatters(3, 1)

    plsc.subcore_barrier()
    pltpu.sync_copy(acc_sh.at[pl.ds(s * SEG, SEG)], zbuf)
    pltpu.sync_copy(zbuf, out_h.at[c, pl.ds(s * SEG, SEG)])


def _alif_update(i_ref, v_ref, a_ref, x_out, v_out, a_out):
    cur = i_ref[0, pl.ds(0, N)] + i_ref[1, pl.ds(0, N)]
    V = v_ref[...]
    A = a_ref[...]
    X = (V - (THR + BETA * A) >= 0.0).astype(jnp.float32)
    x_out[...] = X
    v_out[...] = ALPHA * V * (1.0 - X) + cur
    a_out[...] = RHO * A + X


def kernel(FF, V, a, spike_buffer, edge_w, ff_w, edge_pre, edge_post,
           edge_delay, ff_pre, ff_post):
    sb_flat = spike_buffer.reshape(-1)

    acc = _sc_currents(sb_flat, edge_pre, edge_post, edge_delay, edge_w, FF,
                       ff_pre, ff_post, ff_w)

    shp = jax.ShapeDtypeStruct((N,), jnp.float32)
    X, Vn, an = pl.pallas_call(
        _alif_update,
        out_shape=[shp, shp, shp],
    )(acc, V, a)
    return (X, Vn, an)
